# Initial kernel scaffold; baseline (speedup 1.0000x reference)
import jax
import jax.numpy as jnp
from jax.experimental import pallas as pl


def kernel(q, keys, values):
    raise NotImplementedError("write your pallas kernel here")


# speedup vs baseline: 1.3613x; 1.3613x over previous
import functools

import jax
import jax.numpy as jnp
from jax import lax
from jax.experimental import pallas as pl
from jax.experimental.pallas import tpu as pltpu
from jax.experimental.pallas import tpu_sc as plsc

NQ = 4096
ND = 256
NK = 65536
K = 5

TQ = 256
TK = 1024
NEG = float(jnp.finfo(jnp.float32).min)


def _row_normalize(x):
    n = jnp.sqrt(jnp.sum(x * x, axis=1, keepdims=True))
    return x / jnp.maximum(n, 1e-12)


def _topk_body(q_ref, k_ref, w_ref, i_ref, runv_ref, runi_ref):
    kk = pl.program_id(1)
    nk = pl.num_programs(1)

    @pl.when(kk == 0)
    def _init():
        runv_ref[...] = jnp.full((TQ, 8), NEG, jnp.float32)
        runi_ref[...] = jnp.zeros((TQ, 8), jnp.int32)

    qn = _row_normalize(q_ref[...])
    kn = _row_normalize(k_ref[...])
    sim = lax.dot_general(
        qn, kn, (((1,), (1,)), ((), ())), preferred_element_type=jnp.float32
    )

    iota = lax.broadcasted_iota(jnp.int32, (TQ, TK), 1)
    bvals = []
    bidxs = []
    for _ in range(K):
        m = jnp.max(sim, axis=1, keepdims=True)
        pos = jnp.min(jnp.where(sim == m, iota, TK), axis=1, keepdims=True)
        bvals.append(m)
        bidxs.append(pos + kk * TK)
        sim = jnp.where(iota == pos, NEG, sim)

    cv = jnp.concatenate(
        [runv_ref[...]] + bvals + [jnp.full((TQ, 3), NEG, jnp.float32)], axis=1
    )
    ci = jnp.concatenate(
        [runi_ref[...]] + bidxs + [jnp.zeros((TQ, 3), jnp.int32)], axis=1
    )
    iota16 = lax.broadcasted_iota(jnp.int32, (TQ, 16), 1)
    nv = []
    ni = []
    for _ in range(K):
        m = jnp.max(cv, axis=1, keepdims=True)
        pos = jnp.min(jnp.where(cv == m, iota16, 16), axis=1, keepdims=True)
        nv.append(m)
        ni.append(
            jnp.sum(jnp.where(iota16 == pos, ci, 0), axis=1, keepdims=True)
        )
        cv = jnp.where(iota16 == pos, NEG, cv)

    newv = jnp.concatenate(nv + [jnp.full((TQ, 3), NEG, jnp.float32)], axis=1)
    newi = jnp.concatenate(ni + [jnp.zeros((TQ, 3), jnp.int32)], axis=1)
    runv_ref[...] = newv
    runi_ref[...] = newi

    @pl.when(kk == nk - 1)
    def _finish():
        v = newv[:, 0:K]
        mx = jnp.max(v, axis=1, keepdims=True)
        e = jnp.exp(v - mx)
        w = e / jnp.sum(e, axis=1, keepdims=True)
        w_ref[...] = jnp.concatenate(
            [w, jnp.zeros((TQ, 11), jnp.float32)], axis=1
        )
        i_ref[...] = jnp.concatenate(
            [newi[:, 0:K], jnp.zeros((TQ, 11), jnp.int32)], axis=1
        )


def _topk_tc(q, keys):
    grid = (NQ // TQ, NK // TK)
    return pl.pallas_call(
        _topk_body,
        grid=grid,
        in_specs=[
            pl.BlockSpec((TQ, ND), lambda i, k: (i, 0)),
            pl.BlockSpec((TK, ND), lambda i, k: (k, 0)),
        ],
        out_specs=[
            pl.BlockSpec((TQ, 16), lambda i, k: (i, 0)),
            pl.BlockSpec((TQ, 16), lambda i, k: (i, 0)),
        ],
        out_shape=[
            jax.ShapeDtypeStruct((NQ, 16), jnp.float32),
            jax.ShapeDtypeStruct((NQ, 16), jnp.int32),
        ],
        scratch_shapes=[
            pltpu.VMEM((TQ, 8), jnp.float32),
            pltpu.VMEM((TQ, 8), jnp.int32),
        ],
        compiler_params=pltpu.CompilerParams(
            dimension_semantics=("parallel", "arbitrary"),
        ),
    )(q, keys)


def _gather_sc(values, idxf, w16s):
    info = plsc.get_sparse_core_info()
    nc, ns, nl = info.num_cores, info.num_subcores, info.num_lanes
    nw = nc * ns
    b = idxf.shape[0]
    b_per_w = b // nw
    ch_q = 16
    ch_r = ch_q * K
    n_ch = b_per_w // ch_r
    dsub = ND // nl

    mesh = plsc.VectorSubcoreMesh(core_axis_name="c", subcore_axis_name="s")

    @functools.partial(
        pl.kernel,
        mesh=mesh,
        out_type=jax.ShapeDtypeStruct((NQ, ND), jnp.float32),
        scratch_types=[
            pltpu.VMEM((ch_r,), jnp.int32),
            pltpu.VMEM((ch_q, 16), jnp.float32),
            pltpu.VMEM((ch_r, ND), jnp.float32),
            pltpu.VMEM((ch_q, ND), jnp.float32),
            pltpu.SemaphoreType.DMA,
        ],
        compiler_params=pltpu.CompilerParams(needs_layout_passes=False),
    )
    def sc_kernel(values_hbm, idx_hbm, w_hbm, out_hbm, idx_v, w_v, rows_v,
                  out_v, sem):
        wid = lax.axis_index("s") * nc + lax.axis_index("c")
        base = wid * b_per_w
        qb0 = base // K
        lane_iota = lax.iota(jnp.int32, nl)

        def chunk_body(c, carry):
            rbase = pl.multiple_of(base + c * ch_r, 8)
            qbase = pl.multiple_of(qb0 + c * ch_q, 8)
            pltpu.sync_copy(idx_hbm.at[pl.ds(rbase, ch_r)], idx_v)
            pltpu.sync_copy(w_hbm.at[pl.ds(qbase, ch_q)], w_v)
            pltpu.async_copy(values_hbm.at[idx_v], rows_v, sem).wait()

            def q_body(qi, qcarry):
                wrow = w_v[qi, :]
                for j in range(K):
                    r = qi * K + j
                    wj = jnp.sum(
                        jnp.where(lane_iota == j, wrow, 0.0)
                    )
                    w16 = jnp.broadcast_to(wj, (nl,))
                    for d in range(dsub):
                        seg = rows_v[r, pl.ds(d * nl, nl)] * w16
                        if j == 0:
                            out_v[qi, pl.ds(d * nl, nl)] = seg
                        else:
                            out_v[qi, pl.ds(d * nl, nl)] += seg
                return qcarry

            lax.fori_loop(0, ch_q, q_body, 0)
            pltpu.sync_copy(out_v, out_hbm.at[pl.ds(qbase, ch_q)])
            return carry

        lax.fori_loop(0, n_ch, chunk_body, 0)

    return sc_kernel(values, idxf, w16s)


def kernel(q, keys, values):
    w16s, i16 = _topk_tc(q, keys)
    idxf = i16[:, :K].reshape(-1)
    out = _gather_sc(values, idxf, w16s)
    return out[:, :, None, None]


# speedup vs baseline: 2.8037x; 2.0597x over previous
import functools

import jax
import jax.numpy as jnp
from jax import lax
from jax.experimental import pallas as pl
from jax.experimental.pallas import tpu as pltpu
from jax.experimental.pallas import tpu_sc as plsc

NQ = 4096
ND = 256
NK = 65536
K = 5

TK = 256
NEG = float(jnp.finfo(jnp.float32).min)
BIGF = 1e9


def _row_normalize(x):
    n = jnp.sqrt(jnp.sum(x * x, axis=1, keepdims=True))
    return x / jnp.maximum(n, 1e-12)


def _topk_body(k_ref, q_ref, w_ref, i_ref, qn_ref, runv_ref, runi_ref):
    kk = pl.program_id(0)
    nk = pl.num_programs(0)

    @pl.when(kk == 0)
    def _init():
        qn_ref[...] = _row_normalize(q_ref[...])
        runv_ref[...] = jnp.full((8, NQ), NEG, jnp.float32)
        runi_ref[...] = jnp.zeros((8, NQ), jnp.float32)

    kn = _row_normalize(k_ref[...])
    sim = lax.dot_general(
        kn, qn_ref[...], (((1,), (1,)), ((), ())),
        preferred_element_type=jnp.float32,
    )

    iota = lax.broadcasted_iota(jnp.int32, (TK, NQ), 0).astype(jnp.float32)
    base = (kk * TK).astype(jnp.float32)
    bvals = []
    bidxs = []
    for _ in range(K):
        m = jnp.max(sim, axis=0, keepdims=True)
        cand = jnp.where(sim == m, iota, BIGF)
        pos = jnp.min(cand, axis=0, keepdims=True)
        bvals.append(m)
        bidxs.append(pos + base)
        sim = jnp.where(cand == pos, NEG, sim)

    pad3neg = jnp.full((3, NQ), NEG, jnp.float32)
    pad3f = jnp.zeros((3, NQ), jnp.float32)
    cv = jnp.concatenate([runv_ref[...]] + bvals + [pad3neg], axis=0)
    ci = jnp.concatenate([runi_ref[...]] + bidxs + [pad3f], axis=0)
    iota16 = lax.broadcasted_iota(jnp.int32, (16, NQ), 0).astype(jnp.float32)
    nv = []
    ni = []
    for _ in range(K):
        m = jnp.max(cv, axis=0, keepdims=True)
        pos = jnp.min(jnp.where(cv == m, iota16, BIGF), axis=0, keepdims=True)
        hit = iota16 == pos
        nv.append(m)
        ni.append(jnp.sum(jnp.where(hit, ci, 0.0), axis=0, keepdims=True))
        cv = jnp.where(hit, NEG, cv)

    newv = jnp.concatenate(nv + [pad3neg], axis=0)
    newi = jnp.concatenate(ni + [pad3f], axis=0)
    runv_ref[...] = newv
    runi_ref[...] = newi

    @pl.when(kk == nk - 1)
    def _finish():
        v = newv[0:K, :]
        mx = jnp.max(v, axis=0, keepdims=True)
        e = jnp.exp(v - mx)
        w = e / jnp.sum(e, axis=0, keepdims=True)
        w_ref[...] = jnp.concatenate([w, pad3f], axis=0)
        i_ref[...] = newi.astype(jnp.int32)


def _topk_tc(q, keys):
    grid = (NK // TK,)
    return pl.pallas_call(
        _topk_body,
        grid=grid,
        in_specs=[
            pl.BlockSpec((TK, ND), lambda k: (k, 0)),
            pl.BlockSpec((NQ, ND), lambda k: (0, 0)),
        ],
        out_specs=[
            pl.BlockSpec((8, NQ), lambda k: (0, 0)),
            pl.BlockSpec((8, NQ), lambda k: (0, 0)),
        ],
        out_shape=[
            jax.ShapeDtypeStruct((8, NQ), jnp.float32),
            jax.ShapeDtypeStruct((8, NQ), jnp.int32),
        ],
        scratch_shapes=[
            pltpu.VMEM((NQ, ND), jnp.float32),
            pltpu.VMEM((8, NQ), jnp.float32),
            pltpu.VMEM((8, NQ), jnp.float32),
        ],
        compiler_params=pltpu.CompilerParams(
            dimension_semantics=("arbitrary",),
        ),
    )(keys, q)


def _gather_sc(values, idxf, w16s):
    info = plsc.get_sparse_core_info()
    nc, ns, nl = info.num_cores, info.num_subcores, info.num_lanes
    nw = nc * ns
    b = idxf.shape[0]
    b_per_w = b // nw
    ch_q = 16
    ch_r = ch_q * K
    n_ch = b_per_w // ch_r
    dsub = ND // nl

    mesh = plsc.VectorSubcoreMesh(core_axis_name="c", subcore_axis_name="s")

    @functools.partial(
        pl.kernel,
        mesh=mesh,
        out_type=jax.ShapeDtypeStruct((NQ, ND), jnp.float32),
        scratch_types=[
            pltpu.VMEM((ch_r,), jnp.int32),
            pltpu.VMEM((ch_q, 16), jnp.float32),
            pltpu.VMEM((ch_r, ND), jnp.float32),
            pltpu.VMEM((ch_q, ND), jnp.float32),
            pltpu.SemaphoreType.DMA,
        ],
        compiler_params=pltpu.CompilerParams(needs_layout_passes=False),
    )
    def sc_kernel(values_hbm, idx_hbm, w_hbm, out_hbm, idx_v, w_v, rows_v,
                  out_v, sem):
        wid = lax.axis_index("s") * nc + lax.axis_index("c")
        base = wid * b_per_w
        qb0 = base // K
        lane_iota = lax.iota(jnp.int32, nl)

        def chunk_body(c, carry):
            rbase = pl.multiple_of(base + c * ch_r, 8)
            qbase = pl.multiple_of(qb0 + c * ch_q, 8)
            pltpu.sync_copy(idx_hbm.at[pl.ds(rbase, ch_r)], idx_v)
            pltpu.sync_copy(w_hbm.at[pl.ds(qbase, ch_q)], w_v)
            pltpu.async_copy(values_hbm.at[idx_v], rows_v, sem).wait()

            def q_body(qi, qcarry):
                wrow = w_v[qi, :]
                for j in range(K):
                    r = qi * K + j
                    wj = jnp.sum(
                        jnp.where(lane_iota == j, wrow, 0.0)
                    )
                    w16 = jnp.broadcast_to(wj, (nl,))
                    for d in range(dsub):
                        seg = rows_v[r, pl.ds(d * nl, nl)] * w16
                        if j == 0:
                            out_v[qi, pl.ds(d * nl, nl)] = seg
                        else:
                            out_v[qi, pl.ds(d * nl, nl)] += seg
                return qcarry

            lax.fori_loop(0, ch_q, q_body, 0)
            pltpu.sync_copy(out_v, out_hbm.at[pl.ds(qbase, ch_q)])
            return carry

        lax.fori_loop(0, n_ch, chunk_body, 0)

    return sc_kernel(values, idxf, w16s)


def kernel(q, keys, values):
    w8, i8 = _topk_tc(q, keys)
    idxf = i8[:K, :].T.reshape(-1)
    w16s = jnp.pad(w8[:K, :].T, ((0, 0), (0, 11)))
    out = _gather_sc(values, idxf, w16s)
    return out[:, :, None, None]


# speedup vs baseline: 3.5542x; 1.2677x over previous
import functools

import jax
import jax.numpy as jnp
from jax import lax
from jax.experimental import pallas as pl
from jax.experimental.pallas import tpu as pltpu
from jax.experimental.pallas import tpu_sc as plsc

NQ = 4096
ND = 256
NK = 65536
K = 5

TK = 512
NEG = float(jnp.finfo(jnp.float32).min)
BIGF = 1e9


def _row_normalize(x):
    n = jnp.sqrt(jnp.sum(x * x, axis=1, keepdims=True))
    return x / jnp.maximum(n, 1e-12)


def _topk_body(k_ref, q_ref, w_ref, i_ref, qn_ref, runv_ref, runi_ref):
    kk = pl.program_id(0)
    nk = pl.num_programs(0)

    @pl.when(kk == 0)
    def _init():
        qn_ref[...] = _row_normalize(q_ref[...])
        runv_ref[...] = jnp.full((8, NQ), NEG, jnp.float32)
        runi_ref[...] = jnp.zeros((8, NQ), jnp.float32)

    kn = _row_normalize(k_ref[...])
    sim = lax.dot_general(
        kn, qn_ref[...], (((1,), (1,)), ((), ())),
        preferred_element_type=jnp.float32,
    )

    h = TK // 2
    a = sim[0:h, :]
    b = sim[h:TK, :]
    io = lax.broadcasted_iota(jnp.int32, (h, NQ), 0).astype(jnp.float32)
    ioh = io + float(h)
    ge = a >= b
    wv = jnp.where(ge, a, b)
    lv = jnp.where(ge, b, a)
    wi = jnp.where(ge, io, ioh)
    li = jnp.where(ge, ioh, io)
    base = (kk * TK).astype(jnp.float32)
    bvals = []
    bidxs = []
    for _ in range(K):
        m = jnp.max(wv, axis=0, keepdims=True)
        cand = jnp.where(wv == m, wi, BIGF)
        pos = jnp.min(cand, axis=0, keepdims=True)
        bvals.append(m)
        bidxs.append(pos + base)
        hit = wi == pos
        wv = jnp.where(hit, lv, wv)
        wi = jnp.where(hit, li, wi)
        lv = jnp.where(hit, NEG, lv)

    pad3neg = jnp.full((3, NQ), NEG, jnp.float32)
    pad3f = jnp.zeros((3, NQ), jnp.float32)
    cv = jnp.concatenate([runv_ref[...]] + bvals + [pad3neg], axis=0)
    ci = jnp.concatenate([runi_ref[...]] + bidxs + [pad3f], axis=0)
    iota16 = lax.broadcasted_iota(jnp.int32, (16, NQ), 0).astype(jnp.float32)
    nv = []
    ni = []
    for _ in range(K):
        m = jnp.max(cv, axis=0, keepdims=True)
        pos = jnp.min(jnp.where(cv == m, iota16, BIGF), axis=0, keepdims=True)
        hit = iota16 == pos
        nv.append(m)
        ni.append(jnp.sum(jnp.where(hit, ci, 0.0), axis=0, keepdims=True))
        cv = jnp.where(hit, NEG, cv)

    newv = jnp.concatenate(nv + [pad3neg], axis=0)
    newi = jnp.concatenate(ni + [pad3f], axis=0)
    runv_ref[...] = newv
    runi_ref[...] = newi

    @pl.when(kk == nk - 1)
    def _finish():
        v = newv[0:K, :]
        mx = jnp.max(v, axis=0, keepdims=True)
        e = jnp.exp(v - mx)
        w = e / jnp.sum(e, axis=0, keepdims=True)
        w_ref[...] = jnp.concatenate([w, pad3f], axis=0)
        i_ref[...] = newi.astype(jnp.int32)


def _topk_tc(q, keys):
    grid = (NK // TK,)
    return pl.pallas_call(
        _topk_body,
        grid=grid,
        in_specs=[
            pl.BlockSpec((TK, ND), lambda k: (k, 0)),
            pl.BlockSpec((NQ, ND), lambda k: (0, 0)),
        ],
        out_specs=[
            pl.BlockSpec((8, NQ), lambda k: (0, 0)),
            pl.BlockSpec((8, NQ), lambda k: (0, 0)),
        ],
        out_shape=[
            jax.ShapeDtypeStruct((8, NQ), jnp.float32),
            jax.ShapeDtypeStruct((8, NQ), jnp.int32),
        ],
        scratch_shapes=[
            pltpu.VMEM((NQ, ND), jnp.float32),
            pltpu.VMEM((8, NQ), jnp.float32),
            pltpu.VMEM((8, NQ), jnp.float32),
        ],
        compiler_params=pltpu.CompilerParams(
            dimension_semantics=("arbitrary",),
        ),
    )(keys, q)


def _gather_sc(values, idxf, w16s):
    info = plsc.get_sparse_core_info()
    nc, ns, nl = info.num_cores, info.num_subcores, info.num_lanes
    nw = nc * ns
    b = idxf.shape[0]
    b_per_w = b // nw
    ch_q = 16
    ch_r = ch_q * K
    n_ch = b_per_w // ch_r
    dsub = ND // nl

    mesh = plsc.VectorSubcoreMesh(core_axis_name="c", subcore_axis_name="s")

    @functools.partial(
        pl.kernel,
        mesh=mesh,
        out_type=jax.ShapeDtypeStruct((NQ, ND), jnp.float32),
        scratch_types=[
            pltpu.VMEM((ch_r,), jnp.int32),
            pltpu.VMEM((ch_q, 16), jnp.float32),
            pltpu.VMEM((ch_r, ND), jnp.float32),
            pltpu.VMEM((ch_q, ND), jnp.float32),
            pltpu.SemaphoreType.DMA,
        ],
        compiler_params=pltpu.CompilerParams(needs_layout_passes=False),
    )
    def sc_kernel(values_hbm, idx_hbm, w_hbm, out_hbm, idx_v, w_v, rows_v,
                  out_v, sem):
        wid = lax.axis_index("s") * nc + lax.axis_index("c")
        base = wid * b_per_w
        qb0 = base // K
        lane_iota = lax.iota(jnp.int32, nl)

        def chunk_body(c, carry):
            rbase = pl.multiple_of(base + c * ch_r, 8)
            qbase = pl.multiple_of(qb0 + c * ch_q, 8)
            pltpu.sync_copy(idx_hbm.at[pl.ds(rbase, ch_r)], idx_v)
            pltpu.sync_copy(w_hbm.at[pl.ds(qbase, ch_q)], w_v)
            pltpu.async_copy(values_hbm.at[idx_v], rows_v, sem).wait()

            def q_body(qi, qcarry):
                wrow = w_v[qi, :]
                for j in range(K):
                    r = qi * K + j
                    wj = jnp.sum(
                        jnp.where(lane_iota == j, wrow, 0.0)
                    )
                    w16 = jnp.broadcast_to(wj, (nl,))
                    for d in range(dsub):
                        seg = rows_v[r, pl.ds(d * nl, nl)] * w16
                        if j == 0:
                            out_v[qi, pl.ds(d * nl, nl)] = seg
                        else:
                            out_v[qi, pl.ds(d * nl, nl)] += seg
                return qcarry

            lax.fori_loop(0, ch_q, q_body, 0)
            pltpu.sync_copy(out_v, out_hbm.at[pl.ds(qbase, ch_q)])
            return carry

        lax.fori_loop(0, n_ch, chunk_body, 0)

    return sc_kernel(values, idxf, w16s)


def kernel(q, keys, values):
    w8, i8 = _topk_tc(q, keys)
    idxf = i8[:K, :].T.reshape(-1)
    w16s = jnp.pad(w8[:K, :].T, ((0, 0), (0, 11)))
    out = _gather_sc(values, idxf, w16s)
    return out[:, :, None, None]
